# baseline (speedup 1.0000x reference)
import functools

import jax
import jax.numpy as jnp
from jax import lax
from jax.experimental import pallas as pl
from jax.experimental.pallas import tpu as pltpu
from jax.experimental.pallas import tpu_sc as plsc

NC = 2
NS = 16
LN = 16
BM = 512
SUP = 8
KPT = 4


def _rsqrt_newton(x):
    bits = plsc.bitcast(x, jnp.int32)
    bits = 0x5F3759DF - lax.shift_right_arithmetic(bits, 1)
    y = plsc.bitcast(bits, jnp.float32)
    for _ in range(3):
        y = y * (1.5 - 0.5 * x * y * y)
    return y


def _make_deg_kernel(NP, RA):
    SL = NP // NS
    mesh = plsc.VectorSubcoreMesh(core_axis_name="c", subcore_axis_name="s",
                                  num_cores=NC, num_subcores=NS)

    @functools.partial(
        pl.kernel, mesh=mesh,
        out_type=jax.ShapeDtypeStruct((NC, 2, NP), jnp.float32),
        scratch_types=[
            pltpu.VMEM((RA, 128), jnp.int32),
            pltpu.VMEM((RA, 128), jnp.int32),
            pltpu.VMEM((NP,), jnp.float32),
            pltpu.VMEM((NP,), jnp.float32),
            pltpu.VMEM((2, SL), jnp.float32),
            pltpu.VMEM((2, SL), jnp.float32),
            pltpu.VMEM_SHARED((NS, 2, NP), jnp.float32),
        ],
        compiler_params=pltpu.CompilerParams(needs_layout_passes=False),
    )
    def deg_kernel(src_hbm, dst_hbm, out_hbm, sidx, didx, dego, degi,
                   accv, tmpv, stage):
        cid = lax.axis_index("c")
        sid = lax.axis_index("s")
        pltpu.sync_copy(src_hbm.at[pl.ds(sid * RA, RA)], sidx)
        pltpu.sync_copy(dst_hbm.at[pl.ds(sid * RA, RA)], didx)
        zeros16 = jnp.zeros((LN,), jnp.float32)

        def zbody(i, _):
            dego[pl.ds(i * LN, LN)] = zeros16
            degi[pl.ds(i * LN, LN)] = zeros16
            return 0
        lax.fori_loop(0, NP // LN, zbody, 0)

        ones16 = jnp.full((LN,), 1.0, jnp.float32)

        @plsc.parallel_loop(0, RA, step=1, unroll=2)
        def ebody(j):
            for g in range(128 // LN):
                si = sidx[j, pl.ds(g * LN, LN)]
                plsc.addupdate_scatter(dego, [si], ones16)
                di = didx[j, pl.ds(g * LN, LN)]
                plsc.addupdate_scatter(degi, [di], ones16)

        pltpu.sync_copy(dego, stage.at[sid, 0])
        pltpu.sync_copy(degi, stage.at[sid, 1])
        plsc.subcore_barrier()

        base = sid * SL
        pltpu.sync_copy(stage.at[0, :, pl.ds(base, SL)], accv)
        for p in range(1, NS):
            pltpu.sync_copy(stage.at[p, :, pl.ds(base, SL)], tmpv)

            def abody(i, _):
                for r in range(2):
                    s = pl.ds(i * LN, LN)
                    accv[r, s] = accv[r, s] + tmpv[r, s]
                return 0
            lax.fori_loop(0, SL // LN, abody, 0)

        def nbody(i, _):
            for r in range(2):
                s = pl.ds(i * LN, LN)
                accv[r, s] = _rsqrt_newton(jnp.maximum(accv[r, s], 1.0))
            return 0
        lax.fori_loop(0, SL // LN, nbody, 0)
        pltpu.sync_copy(accv, out_hbm.at[cid, :, pl.ds(base, SL)])

    return deg_kernel


def _make_agg_kernel(NP, NR):
    NSS = NR // SUP
    mesh = plsc.VectorSubcoreMesh(core_axis_name="c", subcore_axis_name="s",
                                  num_cores=NC, num_subcores=NS)

    @functools.partial(
        pl.kernel, mesh=mesh,
        out_type=jax.ShapeDtypeStruct((NC * NS, KPT, NP), jnp.float32),
        scratch_types=[pltpu.VMEM((NP,), jnp.float32)] * (2 * KPT) + [
            pltpu.VMEM((2, SUP, 128), jnp.int32),
            pltpu.VMEM((2, SUP, 128), jnp.int32),
            pltpu.SemaphoreType.DMA,
            pltpu.SemaphoreType.DMA,
        ],
        compiler_params=pltpu.CompilerParams(needs_layout_passes=False),
    )
    def agg_kernel(hs_hbm, src_hbm, dst_hbm, out_hbm, *rest):
        hcol = rest[:KPT]
        acol = rest[KPT:2 * KPT]
        sbuf, dbuf, ssem, dsem = rest[2 * KPT:]
        cid = lax.axis_index("c")
        sid = lax.axis_index("s")
        wid = sid * NC + cid

        for k in range(KPT):
            pltpu.sync_copy(hs_hbm.at[wid, k], hcol[k])

        zeros16 = jnp.zeros((LN,), jnp.float32)

        def zbody(i, _):
            for k in range(KPT):
                acol[k][pl.ds(i * LN, LN)] = zeros16
            return 0
        lax.fori_loop(0, NP // LN, zbody, 0)

        pltpu.sync_copy(src_hbm.at[pl.ds(0, SUP)], sbuf.at[0])
        pltpu.sync_copy(dst_hbm.at[pl.ds(0, SUP)], dbuf.at[0])

        def body(g, _):
            gmod = g % 2

            @pl.when(g > 0)
            def _():
                pltpu.make_async_copy(src_hbm.at[pl.ds(0, SUP)],
                                      sbuf.at[0], ssem).wait()
                pltpu.make_async_copy(dst_hbm.at[pl.ds(0, SUP)],
                                      dbuf.at[0], dsem).wait()

            @pl.when(g < NSS - 1)
            def _():
                off = pl.multiple_of((g + 1) * SUP, SUP)
                nxt = (g + 1) % 2
                pltpu.async_copy(src_hbm.at[pl.ds(off, SUP)],
                                 sbuf.at[nxt], ssem)
                pltpu.async_copy(dst_hbm.at[pl.ds(off, SUP)],
                                 dbuf.at[nxt], dsem)

            @plsc.parallel_loop(0, SUP, step=1, unroll=4)
            def rbody(r):
                for q in range(128 // LN):
                    s = pl.ds(q * LN, LN)
                    sv = sbuf[gmod, r, s]
                    dv = dbuf[gmod, r, s]
                    for k in range(KPT):
                        vals = plsc.load_gather(hcol[k], [sv])
                        plsc.addupdate_scatter(acol[k], [dv], vals)
            return 0
        lax.fori_loop(0, NSS, body, 0)

        for k in range(KPT):
            pltpu.sync_copy(acol[k], out_hbm.at[wid, k])

    return agg_kernel


def _mm_scale_body(x_ref, w_ref, ns_ref, o_ref):
    y = lax.dot_general(w_ref[...], x_ref[...], (((0,), (1,)), ((), ())),
                        preferred_element_type=jnp.float32)
    o_ref[...] = y * ns_ref[...]


def _post_mm_body(a_ref, nd_ref, b_ref, w_ref, ns_ref, o_ref):
    x = jnp.maximum(a_ref[...] * nd_ref[...] + b_ref[...], 0.0)
    y = lax.dot_general(w_ref[...], x, (((0,), (0,)), ((), ())),
                        preferred_element_type=jnp.float32)
    o_ref[...] = y * ns_ref[...]


def _make_readout_body(NN, NB, H, C):
    def readout_body(a_ref, nd_ref, b_ref, wr_ref, o_ref, acc_ref):
        i = pl.program_id(0)
        x = jnp.maximum(a_ref[...] * nd_ref[...] + b_ref[...], 0.0)
        colid = i * BM + lax.broadcasted_iota(jnp.int32, (H, BM), 1)
        x = jnp.where(colid < NN, x, 0.0)
        s = jnp.sum(x, axis=1, keepdims=True)

        @pl.when(i == 0)
        def _():
            acc_ref[...] = s

        @pl.when(i > 0)
        def _():
            acc_ref[...] = acc_ref[...] + s

        @pl.when(i == NB - 1)
        def _():
            o_ref[...] = lax.dot_general(
                acc_ref[...] / NN, wr_ref[...], (((0,), (0,)), ((), ())),
                preferred_element_type=jnp.float32)
    return readout_body


def kernel(feat, edge_index, W0, b0, W1, b1, Wr):
    NN, D = feat.shape
    E = edge_index.shape[1]
    H = W0.shape[1]
    C = Wr.shape[1]

    NP = -(-(NN + 1) // BM) * BM
    EP = -(-E // (NS * 128 * 8)) * (NS * 128 * 8)
    NR = EP // 128
    RA = NR // NS
    NB = NP // BM

    src = edge_index[0]
    dst = edge_index[1]
    padi = jnp.full((EP - E,), NN, jnp.int32)
    src2d = jnp.concatenate([src, padi]).reshape(NR, 128)
    dst2d = jnp.concatenate([dst, padi]).reshape(NR, 128)
    feat_p = jnp.pad(feat, ((0, NP - NN), (0, 0)))

    norms = _make_deg_kernel(NP, RA)(src2d, dst2d)
    ns = norms[0, 0].reshape(1, NP)
    nd = norms[0, 1].reshape(1, NP)

    colT_spec = pl.BlockSpec((H, BM), lambda i: (0, i))
    nrm_spec = pl.BlockSpec((1, BM), lambda i: (0, i))
    w_spec = pl.BlockSpec((D, H), lambda i: (0, 0))
    bT_spec = pl.BlockSpec((H, 1), lambda i: (0, 0))

    hs1 = pl.pallas_call(
        _mm_scale_body,
        grid=(NB,),
        in_specs=[pl.BlockSpec((BM, D), lambda i: (i, 0)), w_spec, nrm_spec],
        out_specs=colT_spec,
        out_shape=jax.ShapeDtypeStruct((H, NP), jnp.float32),
    )(feat_p, W0, ns)

    agg = _make_agg_kernel(NP, NR)
    p1 = agg(hs1.reshape(NC * NS, KPT, NP), src2d, dst2d)

    hs2 = pl.pallas_call(
        _post_mm_body,
        grid=(NB,),
        in_specs=[colT_spec, nrm_spec, bT_spec, w_spec, nrm_spec],
        out_specs=colT_spec,
        out_shape=jax.ShapeDtypeStruct((H, NP), jnp.float32),
    )(p1.reshape(H, NP), nd, b0.reshape(H, 1), W1, ns)

    p2 = agg(hs2.reshape(NC * NS, KPT, NP), src2d, dst2d)

    out = pl.pallas_call(
        _make_readout_body(NN, NB, H, C),
        grid=(NB,),
        in_specs=[colT_spec, nrm_spec, bT_spec,
                  pl.BlockSpec((H, C), lambda i: (0, 0))],
        out_specs=pl.BlockSpec((1, C), lambda i: (0, 0)),
        out_shape=jax.ShapeDtypeStruct((1, C), jnp.float32),
        scratch_shapes=[pltpu.VMEM((H, 1), jnp.float32)],
    )(p2.reshape(H, NP), nd, b1.reshape(H, 1), Wr)

    return out


# speedup vs baseline: 1.1095x; 1.1095x over previous
import functools

import jax
import jax.numpy as jnp
from jax import lax
from jax.experimental import pallas as pl
from jax.experimental.pallas import tpu as pltpu
from jax.experimental.pallas import tpu_sc as plsc

NC = 2
NS = 16
LN = 16
BM = 512
SUP = 8
KPT = 4


def _rsqrt_newton(x):
    bits = plsc.bitcast(x, jnp.int32)
    bits = 0x5F3759DF - lax.shift_right_arithmetic(bits, 1)
    y = plsc.bitcast(bits, jnp.float32)
    for _ in range(3):
        y = y * (1.5 - 0.5 * x * y * y)
    return y


def _make_deg_kernel(NP, RA):
    SL = NP // NS
    mesh = plsc.VectorSubcoreMesh(core_axis_name="c", subcore_axis_name="s",
                                  num_cores=NC, num_subcores=NS)

    @functools.partial(
        pl.kernel, mesh=mesh,
        out_type=jax.ShapeDtypeStruct((NC, 2, NP), jnp.float32),
        scratch_types=[
            pltpu.VMEM((RA, 128), jnp.int32),
            pltpu.VMEM((RA, 128), jnp.int32),
            pltpu.VMEM((NP,), jnp.float32),
            pltpu.VMEM((NP,), jnp.float32),
            pltpu.VMEM((2, SL), jnp.float32),
            pltpu.VMEM((2, SL), jnp.float32),
            pltpu.VMEM_SHARED((NS, 2, NP), jnp.float32),
        ],
        compiler_params=pltpu.CompilerParams(needs_layout_passes=False),
    )
    def deg_kernel(src_hbm, dst_hbm, out_hbm, sidx, didx, dego, degi,
                   accv, tmpv, stage):
        cid = lax.axis_index("c")
        sid = lax.axis_index("s")
        pltpu.sync_copy(src_hbm.at[pl.ds(sid * RA, RA)], sidx)
        pltpu.sync_copy(dst_hbm.at[pl.ds(sid * RA, RA)], didx)
        zeros16 = jnp.zeros((LN,), jnp.float32)

        def zbody(i, _):
            dego[pl.ds(i * LN, LN)] = zeros16
            degi[pl.ds(i * LN, LN)] = zeros16
            return 0
        lax.fori_loop(0, NP // LN, zbody, 0)

        ones16 = jnp.full((LN,), 1.0, jnp.float32)

        @plsc.parallel_loop(0, RA, step=1, unroll=2)
        def ebody(j):
            for g in range(128 // LN):
                si = sidx[j, pl.ds(g * LN, LN)]
                plsc.addupdate_scatter(dego, [si], ones16)
                di = didx[j, pl.ds(g * LN, LN)]
                plsc.addupdate_scatter(degi, [di], ones16)

        pltpu.sync_copy(dego, stage.at[sid, 0])
        pltpu.sync_copy(degi, stage.at[sid, 1])
        plsc.subcore_barrier()

        base = sid * SL
        pltpu.sync_copy(stage.at[0, :, pl.ds(base, SL)], accv)
        for p in range(1, NS):
            pltpu.sync_copy(stage.at[p, :, pl.ds(base, SL)], tmpv)

            def abody(i, _):
                for r in range(2):
                    s = pl.ds(i * LN, LN)
                    accv[r, s] = accv[r, s] + tmpv[r, s]
                return 0
            lax.fori_loop(0, SL // LN, abody, 0)

        def nbody(i, _):
            for r in range(2):
                s = pl.ds(i * LN, LN)
                accv[r, s] = _rsqrt_newton(jnp.maximum(accv[r, s], 1.0))
            return 0
        lax.fori_loop(0, SL // LN, nbody, 0)
        pltpu.sync_copy(accv, out_hbm.at[cid, :, pl.ds(base, SL)])

    return deg_kernel


def _make_agg_kernel(NP, NR):
    NSS = NR // SUP
    mesh = plsc.VectorSubcoreMesh(core_axis_name="c", subcore_axis_name="s",
                                  num_cores=NC, num_subcores=NS)

    @functools.partial(
        pl.kernel, mesh=mesh,
        out_type=jax.ShapeDtypeStruct((NC * NS, KPT, NP), jnp.float32),
        scratch_types=[pltpu.VMEM((NP,), jnp.float32)] * (2 * KPT) + [
            pltpu.VMEM((2, SUP, 128), jnp.int32),
            pltpu.VMEM((2, SUP, 128), jnp.int32),
            pltpu.SemaphoreType.DMA,
            pltpu.SemaphoreType.DMA,
        ],
        compiler_params=pltpu.CompilerParams(needs_layout_passes=False),
    )
    def agg_kernel(hs_hbm, src_hbm, dst_hbm, out_hbm, *rest):
        hcol = rest[:KPT]
        acol = rest[KPT:2 * KPT]
        sbuf, dbuf, ssem, dsem = rest[2 * KPT:]
        cid = lax.axis_index("c")
        sid = lax.axis_index("s")
        wid = sid * NC + cid

        for k in range(KPT):
            pltpu.sync_copy(hs_hbm.at[wid, k], hcol[k])

        zeros16 = jnp.zeros((LN,), jnp.float32)

        def zbody(i, _):
            for k in range(KPT):
                acol[k][pl.ds(i * LN, LN)] = zeros16
            return 0
        lax.fori_loop(0, NP // LN, zbody, 0)

        pltpu.sync_copy(src_hbm.at[pl.ds(0, SUP)], sbuf.at[0])
        pltpu.sync_copy(dst_hbm.at[pl.ds(0, SUP)], dbuf.at[0])

        def body(g, _):
            gmod = g % 2

            @pl.when(g > 0)
            def _():
                pltpu.make_async_copy(src_hbm.at[pl.ds(0, SUP)],
                                      sbuf.at[0], ssem).wait()
                pltpu.make_async_copy(dst_hbm.at[pl.ds(0, SUP)],
                                      dbuf.at[0], dsem).wait()

            @pl.when(g < NSS - 1)
            def _():
                off = pl.multiple_of((g + 1) * SUP, SUP)
                nxt = (g + 1) % 2
                pltpu.async_copy(src_hbm.at[pl.ds(off, SUP)],
                                 sbuf.at[nxt], ssem)
                pltpu.async_copy(dst_hbm.at[pl.ds(off, SUP)],
                                 dbuf.at[nxt], dsem)

            @plsc.parallel_loop(0, SUP, step=1, unroll=2)
            def rbody(r):
                for q in range(128 // LN):
                    s = pl.ds(q * LN, LN)
                    sv = sbuf[gmod, r, s]
                    dv = dbuf[gmod, r, s]
                    for k in range(KPT):
                        vals = plsc.load_gather(hcol[k], [sv])
                        plsc.addupdate_scatter(acol[k], [dv], vals)
            return 0
        lax.fori_loop(0, NSS, body, 0)

        for k in range(KPT):
            pltpu.sync_copy(acol[k], out_hbm.at[wid, k])

    return agg_kernel


def _mm_scale_body(x_ref, w_ref, ns_ref, o_ref):
    y = lax.dot_general(w_ref[...], x_ref[...], (((0,), (1,)), ((), ())),
                        preferred_element_type=jnp.float32)
    o_ref[...] = y * ns_ref[...]


def _post_mm_body(a_ref, nd_ref, b_ref, w_ref, ns_ref, o_ref):
    x = jnp.maximum(a_ref[...] * nd_ref[...] + b_ref[...], 0.0)
    y = lax.dot_general(w_ref[...], x, (((0,), (0,)), ((), ())),
                        preferred_element_type=jnp.float32)
    o_ref[...] = y * ns_ref[...]


def _make_readout_body(NN, NB, H, C):
    def readout_body(a_ref, nd_ref, b_ref, wr_ref, o_ref, acc_ref):
        i = pl.program_id(0)
        x = jnp.maximum(a_ref[...] * nd_ref[...] + b_ref[...], 0.0)
        colid = i * BM + lax.broadcasted_iota(jnp.int32, (H, BM), 1)
        x = jnp.where(colid < NN, x, 0.0)
        s = jnp.sum(x, axis=1, keepdims=True)

        @pl.when(i == 0)
        def _():
            acc_ref[...] = s

        @pl.when(i > 0)
        def _():
            acc_ref[...] = acc_ref[...] + s

        @pl.when(i == NB - 1)
        def _():
            o_ref[...] = lax.dot_general(
                acc_ref[...] / NN, wr_ref[...], (((0,), (0,)), ((), ())),
                preferred_element_type=jnp.float32)
    return readout_body


def kernel(feat, edge_index, W0, b0, W1, b1, Wr):
    NN, D = feat.shape
    E = edge_index.shape[1]
    H = W0.shape[1]
    C = Wr.shape[1]

    NP = -(-(NN + 1) // BM) * BM
    EP = -(-E // (NS * 128 * 8)) * (NS * 128 * 8)
    NR = EP // 128
    RA = NR // NS
    NB = NP // BM

    src = edge_index[0]
    dst = edge_index[1]
    padi = jnp.full((EP - E,), NN, jnp.int32)
    src2d = jnp.concatenate([src, padi]).reshape(NR, 128)
    dst2d = jnp.concatenate([dst, padi]).reshape(NR, 128)
    feat_p = jnp.pad(feat, ((0, NP - NN), (0, 0)))

    norms = _make_deg_kernel(NP, RA)(src2d, dst2d)
    ns = norms[0, 0].reshape(1, NP)
    nd = norms[0, 1].reshape(1, NP)

    colT_spec = pl.BlockSpec((H, BM), lambda i: (0, i))
    nrm_spec = pl.BlockSpec((1, BM), lambda i: (0, i))
    w_spec = pl.BlockSpec((D, H), lambda i: (0, 0))
    bT_spec = pl.BlockSpec((H, 1), lambda i: (0, 0))

    hs1 = pl.pallas_call(
        _mm_scale_body,
        grid=(NB,),
        in_specs=[pl.BlockSpec((BM, D), lambda i: (i, 0)), w_spec, nrm_spec],
        out_specs=colT_spec,
        out_shape=jax.ShapeDtypeStruct((H, NP), jnp.float32),
    )(feat_p, W0, ns)

    agg = _make_agg_kernel(NP, NR)
    p1 = agg(hs1.reshape(NC * NS, KPT, NP), src2d, dst2d)

    hs2 = pl.pallas_call(
        _post_mm_body,
        grid=(NB,),
        in_specs=[colT_spec, nrm_spec, bT_spec, w_spec, nrm_spec],
        out_specs=colT_spec,
        out_shape=jax.ShapeDtypeStruct((H, NP), jnp.float32),
    )(p1.reshape(H, NP), nd, b0.reshape(H, 1), W1, ns)

    p2 = agg(hs2.reshape(NC * NS, KPT, NP), src2d, dst2d)

    out = pl.pallas_call(
        _make_readout_body(NN, NB, H, C),
        grid=(NB,),
        in_specs=[colT_spec, nrm_spec, bT_spec,
                  pl.BlockSpec((H, C), lambda i: (0, 0))],
        out_specs=pl.BlockSpec((1, C), lambda i: (0, 0)),
        out_shape=jax.ShapeDtypeStruct((1, C), jnp.float32),
        scratch_shapes=[pltpu.VMEM((H, 1), jnp.float32)],
    )(p2.reshape(H, NP), nd, b1.reshape(H, 1), Wr)

    return out


# speedup vs baseline: 1.2114x; 1.0918x over previous
import functools

import jax
import jax.numpy as jnp
from jax import lax
from jax.experimental import pallas as pl
from jax.experimental.pallas import tpu as pltpu
from jax.experimental.pallas import tpu_sc as plsc

NC = 2
NS = 16
NW = NC * NS
LN = 16
BM = 512
SUP = 16
KPT = 4


def _make_deg_kernel(NP, RA):
    mesh = plsc.VectorSubcoreMesh(core_axis_name="c", subcore_axis_name="s",
                                  num_cores=NC, num_subcores=NS)

    @functools.partial(
        pl.kernel, mesh=mesh,
        out_type=jax.ShapeDtypeStruct((NW, 2, NP), jnp.float32),
        scratch_types=[
            pltpu.VMEM((RA, 128), jnp.int32),
            pltpu.VMEM((RA, 128), jnp.int32),
            pltpu.VMEM((NP,), jnp.float32),
            pltpu.VMEM((NP,), jnp.float32),
        ],
        compiler_params=pltpu.CompilerParams(needs_layout_passes=False),
    )
    def deg_kernel(src_hbm, dst_hbm, out_hbm, sidx, didx, dego, degi):
        cid = lax.axis_index("c")
        sid = lax.axis_index("s")
        wid = sid * NC + cid
        pltpu.sync_copy(src_hbm.at[pl.ds(wid * RA, RA)], sidx)
        pltpu.sync_copy(dst_hbm.at[pl.ds(wid * RA, RA)], didx)
        zeros16 = jnp.zeros((LN,), jnp.float32)

        @plsc.parallel_loop(0, NP // LN, step=1, unroll=4)
        def zbody(i):
            dego[pl.ds(i * LN, LN)] = zeros16
            degi[pl.ds(i * LN, LN)] = zeros16

        ones16 = jnp.full((LN,), 1.0, jnp.float32)

        @plsc.parallel_loop(0, RA, step=1, unroll=2)
        def ebody(j):
            for g in range(128 // LN):
                si = sidx[j, pl.ds(g * LN, LN)]
                plsc.addupdate_scatter(dego, [si], ones16)
                di = didx[j, pl.ds(g * LN, LN)]
                plsc.addupdate_scatter(degi, [di], ones16)

        pltpu.sync_copy(dego, out_hbm.at[wid, 0])
        pltpu.sync_copy(degi, out_hbm.at[wid, 1])

    return deg_kernel


def _make_agg_kernel(NP, NR):
    NSS = NR // SUP
    mesh = plsc.VectorSubcoreMesh(core_axis_name="c", subcore_axis_name="s",
                                  num_cores=NC, num_subcores=NS)

    @functools.partial(
        pl.kernel, mesh=mesh,
        out_type=jax.ShapeDtypeStruct((NW, KPT, NP), jnp.float32),
        scratch_types=[pltpu.VMEM((NP,), jnp.float32)] * (2 * KPT) + [
            pltpu.VMEM((2, SUP, 128), jnp.int32),
            pltpu.VMEM((2, SUP, 128), jnp.int32),
            pltpu.SemaphoreType.DMA,
            pltpu.SemaphoreType.DMA,
        ],
        compiler_params=pltpu.CompilerParams(needs_layout_passes=False),
    )
    def agg_kernel(hs_hbm, src_hbm, dst_hbm, out_hbm, *rest):
        hcol = rest[:KPT]
        acol = rest[KPT:2 * KPT]
        sbuf, dbuf, ssem, dsem = rest[2 * KPT:]
        cid = lax.axis_index("c")
        sid = lax.axis_index("s")
        wid = sid * NC + cid

        for k in range(KPT):
            pltpu.sync_copy(hs_hbm.at[wid, k], hcol[k])

        zeros16 = jnp.zeros((LN,), jnp.float32)

        @plsc.parallel_loop(0, NP // LN, step=1, unroll=4)
        def zbody(i):
            for k in range(KPT):
                acol[k][pl.ds(i * LN, LN)] = zeros16

        pltpu.sync_copy(src_hbm.at[pl.ds(0, SUP)], sbuf.at[0])
        pltpu.sync_copy(dst_hbm.at[pl.ds(0, SUP)], dbuf.at[0])

        def body(g, _):
            gmod = g % 2

            @pl.when(g > 0)
            def _():
                pltpu.make_async_copy(src_hbm.at[pl.ds(0, SUP)],
                                      sbuf.at[0], ssem).wait()
                pltpu.make_async_copy(dst_hbm.at[pl.ds(0, SUP)],
                                      dbuf.at[0], dsem).wait()

            @pl.when(g < NSS - 1)
            def _():
                off = pl.multiple_of((g + 1) * SUP, SUP)
                nxt = (g + 1) % 2
                pltpu.async_copy(src_hbm.at[pl.ds(off, SUP)],
                                 sbuf.at[nxt], ssem)
                pltpu.async_copy(dst_hbm.at[pl.ds(off, SUP)],
                                 dbuf.at[nxt], dsem)

            @plsc.parallel_loop(0, SUP, step=1, unroll=2)
            def rbody(r):
                for q in range(128 // LN):
                    s = pl.ds(q * LN, LN)
                    sv = sbuf[gmod, r, s]
                    dv = dbuf[gmod, r, s]
                    for k in range(KPT):
                        vals = plsc.load_gather(hcol[k], [sv])
                        plsc.addupdate_scatter(acol[k], [dv], vals)
            return 0
        lax.fori_loop(0, NSS, body, 0)

        for k in range(KPT):
            pltpu.sync_copy(acol[k], out_hbm.at[wid, k])

    return agg_kernel


def _norms_from_parts(dp):
    deg = jnp.sum(dp, axis=0)
    nrm = lax.rsqrt(jnp.maximum(deg, 1.0))
    return nrm[0:1, :], nrm[1:2, :]


def _mm_scale_body(x_ref, w_ref, dp_ref, o_ref):
    ns, _ = _norms_from_parts(dp_ref[...])
    y = lax.dot_general(w_ref[...], x_ref[...], (((0,), (1,)), ((), ())),
                        preferred_element_type=jnp.float32)
    o_ref[...] = y * ns


def _post_mm_body(a_ref, dp_ref, b_ref, w_ref, o_ref):
    ns, nd = _norms_from_parts(dp_ref[...])
    x = jnp.maximum(a_ref[...] * nd + b_ref[...], 0.0)
    y = lax.dot_general(w_ref[...], x, (((0,), (0,)), ((), ())),
                        preferred_element_type=jnp.float32)
    o_ref[...] = y * ns


def _make_readout_body(NN, NB, H, C):
    def readout_body(a_ref, dp_ref, b_ref, wr_ref, o_ref, acc_ref):
        i = pl.program_id(0)
        _, nd = _norms_from_parts(dp_ref[...])
        x = jnp.maximum(a_ref[...] * nd + b_ref[...], 0.0)
        colid = i * BM + lax.broadcasted_iota(jnp.int32, (H, BM), 1)
        x = jnp.where(colid < NN, x, 0.0)
        s = jnp.sum(x, axis=1, keepdims=True)

        @pl.when(i == 0)
        def _():
            acc_ref[...] = s

        @pl.when(i > 0)
        def _():
            acc_ref[...] = acc_ref[...] + s

        @pl.when(i == NB - 1)
        def _():
            o_ref[...] = lax.dot_general(
                acc_ref[...] / NN, wr_ref[...], (((0,), (0,)), ((), ())),
                preferred_element_type=jnp.float32)
    return readout_body


def kernel(feat, edge_index, W0, b0, W1, b1, Wr):
    NN, D = feat.shape
    E = edge_index.shape[1]
    H = W0.shape[1]
    C = Wr.shape[1]

    NP = -(-(NN + 1) // BM) * BM
    EP = -(-E // (NW * 128 * 16)) * (NW * 128 * 16)
    NR = EP // 128
    RA = NR // NW
    NB = NP // BM

    src = edge_index[0]
    dst = edge_index[1]
    padi = jnp.full((EP - E,), NN, jnp.int32)
    src2d = jnp.concatenate([src, padi]).reshape(NR, 128)
    dst2d = jnp.concatenate([dst, padi]).reshape(NR, 128)
    feat_p = jnp.pad(feat, ((0, NP - NN), (0, 0)))

    dparts = _make_deg_kernel(NP, RA)(src2d, dst2d)

    colT_spec = pl.BlockSpec((H, BM), lambda i: (0, i))
    dp_spec = pl.BlockSpec((NW, 2, BM), lambda i: (0, 0, i))
    w_spec = pl.BlockSpec((D, H), lambda i: (0, 0))
    bT_spec = pl.BlockSpec((H, 1), lambda i: (0, 0))

    hs1 = pl.pallas_call(
        _mm_scale_body,
        grid=(NB,),
        in_specs=[pl.BlockSpec((BM, D), lambda i: (i, 0)), w_spec, dp_spec],
        out_specs=colT_spec,
        out_shape=jax.ShapeDtypeStruct((H, NP), jnp.float32),
    )(feat_p, W0, dparts)

    agg = _make_agg_kernel(NP, NR)
    p1 = agg(hs1.reshape(NW, KPT, NP), src2d, dst2d)

    hs2 = pl.pallas_call(
        _post_mm_body,
        grid=(NB,),
        in_specs=[colT_spec, dp_spec, bT_spec, w_spec],
        out_specs=colT_spec,
        out_shape=jax.ShapeDtypeStruct((H, NP), jnp.float32),
    )(p1.reshape(H, NP), dparts, b0.reshape(H, 1), W1)

    p2 = agg(hs2.reshape(NW, KPT, NP), src2d, dst2d)

    out = pl.pallas_call(
        _make_readout_body(NN, NB, H, C),
        grid=(NB,),
        in_specs=[colT_spec, dp_spec, bT_spec,
                  pl.BlockSpec((H, C), lambda i: (0, 0))],
        out_specs=pl.BlockSpec((1, C), lambda i: (0, 0)),
        out_shape=jax.ShapeDtypeStruct((1, C), jnp.float32),
        scratch_shapes=[pltpu.VMEM((H, 1), jnp.float32)],
    )(p2.reshape(H, NP), dparts, b1.reshape(H, 1), Wr)

    return out


# speedup vs baseline: 1.2531x; 1.0344x over previous
import functools

import jax
import jax.numpy as jnp
from jax import lax
from jax.experimental import pallas as pl
from jax.experimental.pallas import tpu as pltpu
from jax.experimental.pallas import tpu_sc as plsc

NC = 2
NS = 16
NW = NC * NS
LN = 16
BM = 512
SUP = 16
KPT = 4


def _make_deg_kernel(NP, RA):
    mesh = plsc.VectorSubcoreMesh(core_axis_name="c", subcore_axis_name="s",
                                  num_cores=NC, num_subcores=NS)

    @functools.partial(
        pl.kernel, mesh=mesh,
        out_type=jax.ShapeDtypeStruct((NW, 2, NP), jnp.float32),
        scratch_types=[
            pltpu.VMEM((RA, 128), jnp.int32),
            pltpu.VMEM((RA, 128), jnp.int32),
            pltpu.VMEM((NP,), jnp.float32),
            pltpu.VMEM((NP,), jnp.float32),
        ],
        compiler_params=pltpu.CompilerParams(needs_layout_passes=False),
    )
    def deg_kernel(src_hbm, dst_hbm, out_hbm, sidx, didx, dego, degi):
        cid = lax.axis_index("c")
        sid = lax.axis_index("s")
        wid = sid * NC + cid
        pltpu.sync_copy(src_hbm.at[pl.ds(wid * RA, RA)], sidx)
        pltpu.sync_copy(dst_hbm.at[pl.ds(wid * RA, RA)], didx)
        zeros16 = jnp.zeros((LN,), jnp.float32)

        @plsc.parallel_loop(0, NP // LN, step=1, unroll=4)
        def zbody(i):
            dego[pl.ds(i * LN, LN)] = zeros16
            degi[pl.ds(i * LN, LN)] = zeros16

        ones16 = jnp.full((LN,), 1.0, jnp.float32)

        @plsc.parallel_loop(0, RA, step=1, unroll=2)
        def ebody(j):
            for g in range(128 // LN):
                si = sidx[j, pl.ds(g * LN, LN)]
                plsc.addupdate_scatter(dego, [si], ones16)
                di = didx[j, pl.ds(g * LN, LN)]
                plsc.addupdate_scatter(degi, [di], ones16)

        pltpu.sync_copy(dego, out_hbm.at[wid, 0])
        pltpu.sync_copy(degi, out_hbm.at[wid, 1])

    return deg_kernel


def _make_agg_kernel(NP, NR):
    NSS = NR // SUP
    mesh = plsc.VectorSubcoreMesh(core_axis_name="c", subcore_axis_name="s",
                                  num_cores=NC, num_subcores=NS)

    @functools.partial(
        pl.kernel, mesh=mesh,
        out_type=jax.ShapeDtypeStruct((NW, KPT, NP), jnp.float32),
        scratch_types=[pltpu.VMEM((NP,), jnp.float32)] * (2 * KPT) + [
            pltpu.VMEM((2, SUP, 128), jnp.int32),
            pltpu.VMEM((2, SUP, 128), jnp.int32),
            pltpu.SemaphoreType.DMA,
            pltpu.SemaphoreType.DMA,
        ],
        compiler_params=pltpu.CompilerParams(needs_layout_passes=False),
    )
    def agg_kernel(hs_hbm, src_hbm, dst_hbm, out_hbm, *rest):
        hcol = rest[:KPT]
        acol = rest[KPT:2 * KPT]
        sbuf, dbuf, ssem, dsem = rest[2 * KPT:]
        cid = lax.axis_index("c")
        sid = lax.axis_index("s")
        wid = sid * NC + cid

        for k in range(KPT):
            pltpu.sync_copy(hs_hbm.at[wid, k], hcol[k])

        zeros16 = jnp.zeros((LN,), jnp.float32)

        @plsc.parallel_loop(0, NP // LN, step=1, unroll=4)
        def zbody(i):
            for k in range(KPT):
                acol[k][pl.ds(i * LN, LN)] = zeros16

        pltpu.sync_copy(src_hbm.at[pl.ds(0, SUP)], sbuf.at[0])
        pltpu.sync_copy(dst_hbm.at[pl.ds(0, SUP)], dbuf.at[0])

        def body(g, _):
            gmod = g % 2

            @pl.when(g > 0)
            def _():
                pltpu.make_async_copy(src_hbm.at[pl.ds(0, SUP)],
                                      sbuf.at[0], ssem).wait()
                pltpu.make_async_copy(dst_hbm.at[pl.ds(0, SUP)],
                                      dbuf.at[0], dsem).wait()

            @pl.when(g < NSS - 1)
            def _():
                off = pl.multiple_of((g + 1) * SUP, SUP)
                nxt = (g + 1) % 2
                pltpu.async_copy(src_hbm.at[pl.ds(off, SUP)],
                                 sbuf.at[nxt], ssem)
                pltpu.async_copy(dst_hbm.at[pl.ds(off, SUP)],
                                 dbuf.at[nxt], dsem)

            @plsc.parallel_loop(0, SUP * (128 // LN), step=1, unroll=4)
            def gbody(t):
                r = lax.shift_right_logical(t, 3)
                s = pl.ds(lax.shift_left(lax.bitwise_and(t, 7), 4), LN)
                sv = sbuf[gmod, r, s]
                dv = dbuf[gmod, r, s]
                for k in range(KPT):
                    vals = plsc.load_gather(hcol[k], [sv])
                    plsc.addupdate_scatter(acol[k], [dv], vals)
            return 0
        lax.fori_loop(0, NSS, body, 0)

        for k in range(KPT):
            pltpu.sync_copy(acol[k], out_hbm.at[wid, k])

    return agg_kernel


def _norms_from_parts(dp):
    deg = jnp.sum(dp, axis=0)
    nrm = lax.rsqrt(jnp.maximum(deg, 1.0))
    return nrm[0:1, :], nrm[1:2, :]


def _mm_scale_body(x_ref, w_ref, dp_ref, o_ref):
    ns, _ = _norms_from_parts(dp_ref[...])
    y = lax.dot_general(w_ref[...], x_ref[...], (((0,), (1,)), ((), ())),
                        preferred_element_type=jnp.float32)
    o_ref[...] = y * ns


def _post_mm_body(a_ref, dp_ref, b_ref, w_ref, o_ref):
    ns, nd = _norms_from_parts(dp_ref[...])
    x = jnp.maximum(a_ref[...] * nd + b_ref[...], 0.0)
    y = lax.dot_general(w_ref[...], x, (((0,), (0,)), ((), ())),
                        preferred_element_type=jnp.float32)
    o_ref[...] = y * ns


def _make_readout_body(NN, NB, H, C):
    def readout_body(a_ref, dp_ref, b_ref, wr_ref, o_ref, acc_ref):
        i = pl.program_id(0)
        _, nd = _norms_from_parts(dp_ref[...])
        x = jnp.maximum(a_ref[...] * nd + b_ref[...], 0.0)
        colid = i * BM + lax.broadcasted_iota(jnp.int32, (H, BM), 1)
        x = jnp.where(colid < NN, x, 0.0)
        s = jnp.sum(x, axis=1, keepdims=True)

        @pl.when(i == 0)
        def _():
            acc_ref[...] = s

        @pl.when(i > 0)
        def _():
            acc_ref[...] = acc_ref[...] + s

        @pl.when(i == NB - 1)
        def _():
            o_ref[...] = lax.dot_general(
                acc_ref[...] / NN, wr_ref[...], (((0,), (0,)), ((), ())),
                preferred_element_type=jnp.float32)
    return readout_body


def kernel(feat, edge_index, W0, b0, W1, b1, Wr):
    NN, D = feat.shape
    E = edge_index.shape[1]
    H = W0.shape[1]
    C = Wr.shape[1]

    NP = -(-(NN + 1) // BM) * BM
    EP = -(-E // (NW * 128 * 16)) * (NW * 128 * 16)
    NR = EP // 128
    RA = NR // NW
    NB = NP // BM

    src = edge_index[0]
    dst = edge_index[1]
    padi = jnp.full((EP - E,), NN, jnp.int32)
    src2d = jnp.concatenate([src, padi]).reshape(NR, 128)
    dst2d = jnp.concatenate([dst, padi]).reshape(NR, 128)
    feat_p = jnp.pad(feat, ((0, NP - NN), (0, 0)))

    dparts = _make_deg_kernel(NP, RA)(src2d, dst2d)

    colT_spec = pl.BlockSpec((H, BM), lambda i: (0, i))
    dp_spec = pl.BlockSpec((NW, 2, BM), lambda i: (0, 0, i))
    w_spec = pl.BlockSpec((D, H), lambda i: (0, 0))
    bT_spec = pl.BlockSpec((H, 1), lambda i: (0, 0))

    hs1 = pl.pallas_call(
        _mm_scale_body,
        grid=(NB,),
        in_specs=[pl.BlockSpec((BM, D), lambda i: (i, 0)), w_spec, dp_spec],
        out_specs=colT_spec,
        out_shape=jax.ShapeDtypeStruct((H, NP), jnp.float32),
    )(feat_p, W0, dparts)

    agg = _make_agg_kernel(NP, NR)
    p1 = agg(hs1.reshape(NW, KPT, NP), src2d, dst2d)

    hs2 = pl.pallas_call(
        _post_mm_body,
        grid=(NB,),
        in_specs=[colT_spec, dp_spec, bT_spec, w_spec],
        out_specs=colT_spec,
        out_shape=jax.ShapeDtypeStruct((H, NP), jnp.float32),
    )(p1.reshape(H, NP), dparts, b0.reshape(H, 1), W1)

    p2 = agg(hs2.reshape(NW, KPT, NP), src2d, dst2d)

    out = pl.pallas_call(
        _make_readout_body(NN, NB, H, C),
        grid=(NB,),
        in_specs=[colT_spec, dp_spec, bT_spec,
                  pl.BlockSpec((H, C), lambda i: (0, 0))],
        out_specs=pl.BlockSpec((1, C), lambda i: (0, 0)),
        out_shape=jax.ShapeDtypeStruct((1, C), jnp.float32),
        scratch_shapes=[pltpu.VMEM((H, 1), jnp.float32)],
    )(p2.reshape(H, NP), dparts, b1.reshape(H, 1), Wr)

    return out
